# Initial kernel scaffold; baseline (speedup 1.0000x reference)
import functools

import jax
import jax.numpy as jnp
from jax import lax
from jax.experimental import pallas as pl
from jax.experimental.pallas import tpu as pltpu
from jax.experimental.pallas import tpu_sc as plsc

N = 10000
E = 320000
D = 128

NC = 2
NS = 16
L = 16
NW = NC * NS
EPW = E // NW
NG = EPW // L
NBUF = 5
ROWS_PT = N // NS

_SC_MESH = plsc.VectorSubcoreMesh(core_axis_name="c", subcore_axis_name="s")


def _worker_id():
    return lax.axis_index("c") * NS + lax.axis_index("s")


def _sc_deg_body(dst_hbm, out_hbm, dstbuf, ones_v, zbuf, deg_sh, sem):
    c = lax.axis_index("c")
    s = lax.axis_index("s")
    wid = _worker_id()

    def _zero_row(i, _):
        zbuf[i, :] = jnp.zeros((L,), jnp.float32)
        return 0

    lax.fori_loop(0, ROWS_PT, _zero_row, 0)

    def _one_row(i, _):
        ones_v[i, :] = jnp.ones((L,), jnp.float32)
        return 0

    lax.fori_loop(0, L, _one_row, 0)

    pltpu.sync_copy(zbuf, deg_sh.at[pl.ds(s * ROWS_PT, ROWS_PT)])
    pltpu.sync_copy(dst_hbm.at[wid], dstbuf)
    plsc.subcore_barrier()

    def _start(j, b):
        idx = dstbuf[pl.ds(j * L, L)]
        pltpu.async_copy(ones_v, deg_sh.at[idx], sem.at[b], add=True)

    def _wait(b):
        pltpu.make_async_copy(ones_v, deg_sh.at[pl.ds(0, L)], sem.at[b]).wait()

    for b in range(NBUF):
        _start(b, b)

    def _loop(jo, _):
        for b in range(NBUF):
            j = jo * NBUF + b
            _wait(b)
            _start(j + NBUF, b)
        return 0

    lax.fori_loop(0, NG // NBUF - 1, _loop, 0)
    for b in range(NBUF):
        _wait(b)

    plsc.subcore_barrier()
    pltpu.sync_copy(deg_sh.at[pl.ds(s * ROWS_PT, ROWS_PT)],
                    out_hbm.at[c, pl.ds(s * ROWS_PT, ROWS_PT)])


_sc_deg = pl.kernel(
    _sc_deg_body,
    out_type=jax.ShapeDtypeStruct((NC, N, L), jnp.float32),
    mesh=_SC_MESH,
    scratch_types=dict(
        dstbuf=pltpu.VMEM((EPW,), jnp.int32),
        ones_v=pltpu.VMEM((L, L), jnp.float32),
        zbuf=pltpu.VMEM((ROWS_PT, L), jnp.float32),
        deg_sh=pltpu.VMEM_SHARED((N, L), jnp.float32),
        sem=pltpu.SemaphoreType.DMA((NBUF,)),
    ),
)


def _sc_agg_body(hp_hbm, src_hbm, dst_hbm, out_hbm,
                 srcbuf, dstbuf, rowbuf, zbuf, acc_sh, gsem):
    c = lax.axis_index("c")
    s = lax.axis_index("s")
    wid = _worker_id()

    def _zero_row(i, _):
        for k in range(D // L):
            zbuf[i, pl.ds(k * L, L)] = jnp.zeros((L,), jnp.float32)
        return 0

    lax.fori_loop(0, 25, _zero_row, 0)

    def _zero_acc(t, _):
        pltpu.sync_copy(zbuf, acc_sh.at[pl.ds(s * ROWS_PT + t * 25, 25)])
        return 0

    lax.fori_loop(0, ROWS_PT // 25, _zero_acc, 0)

    pltpu.sync_copy(src_hbm.at[wid], srcbuf)
    pltpu.sync_copy(dst_hbm.at[wid], dstbuf)
    plsc.subcore_barrier()

    def _start(j, b):
        idx = srcbuf[pl.ds(j * L, L)]
        pltpu.async_copy(hp_hbm.at[idx], rowbuf.at[b], gsem.at[b])

    def _wait(b):
        pltpu.make_async_copy(hp_hbm.at[pl.ds(0, L)], rowbuf.at[b],
                              gsem.at[b]).wait()

    def _scatter(j, b):
        idx = dstbuf[pl.ds(j * L, L)]
        pltpu.sync_copy(rowbuf.at[b], acc_sh.at[idx], add=True)

    for b in range(NBUF):
        _start(b, b)

    def _loop(jo, _):
        for b in range(NBUF):
            j = jo * NBUF + b
            _wait(b)
            _scatter(j, b)
            _start(j + NBUF, b)
        return 0

    lax.fori_loop(0, NG // NBUF - 1, _loop, 0)
    for b in range(NBUF):
        j = NG - NBUF + b
        _wait(b)
        _scatter(j, b)

    plsc.subcore_barrier()
    pltpu.sync_copy(acc_sh.at[pl.ds(s * ROWS_PT, ROWS_PT)],
                    out_hbm.at[c, pl.ds(s * ROWS_PT, ROWS_PT)])


_sc_agg = pl.kernel(
    _sc_agg_body,
    out_type=jax.ShapeDtypeStruct((NC, N, D), jnp.float32),
    mesh=_SC_MESH,
    scratch_types=dict(
        srcbuf=pltpu.VMEM((EPW,), jnp.int32),
        dstbuf=pltpu.VMEM((EPW,), jnp.int32),
        rowbuf=pltpu.VMEM((NBUF, L, D), jnp.float32),
        zbuf=pltpu.VMEM((25, D), jnp.float32),
        acc_sh=pltpu.VMEM_SHARED((N, D), jnp.float32),
        gsem=pltpu.SemaphoreType.DMA((NBUF,)),
    ),
)


def _sc_dec_body(z_hbm, e0_hbm, e1_hbm, out_hbm,
                 e0buf, e1buf, srows, drows, outbuf, sem0, sem1):
    wid = _worker_id()

    pltpu.sync_copy(e0_hbm.at[wid], e0buf)
    pltpu.sync_copy(e1_hbm.at[wid], e1buf)

    rows16 = lax.iota(jnp.int32, L)

    def _start(j, b):
        i0 = e0buf[pl.ds(j * L, L)]
        i1 = e1buf[pl.ds(j * L, L)]
        pltpu.async_copy(z_hbm.at[i0], srows.at[b], sem0.at[b])
        pltpu.async_copy(z_hbm.at[i1], drows.at[b], sem1.at[b])

    def _wait(b):
        pltpu.make_async_copy(z_hbm.at[pl.ds(0, L)], srows.at[b],
                              sem0.at[b]).wait()
        pltpu.make_async_copy(z_hbm.at[pl.ds(0, L)], drows.at[b],
                              sem1.at[b]).wait()

    def _dot(j, b):
        acc = jnp.zeros((L,), jnp.float32)
        sref = srows.at[b]
        dref = drows.at[b]
        for col in range(D):
            cidx = jnp.full((L,), col, jnp.int32)
            sv = plsc.load_gather(sref, [rows16, cidx])
            dv = plsc.load_gather(dref, [rows16, cidx])
            acc = acc + sv * dv
        outbuf[pl.ds(j * L, L)] = acc

    for b in range(NBUF):
        _start(b, b)

    def _loop(jo, _):
        for b in range(NBUF):
            j = jo * NBUF + b
            _wait(b)
            _dot(j, b)
            _start(j + NBUF, b)
        return 0

    lax.fori_loop(0, NG // NBUF - 1, _loop, 0)
    for b in range(NBUF):
        j = NG - NBUF + b
        _wait(b)
        _dot(j, b)

    pltpu.sync_copy(outbuf, out_hbm.at[wid])


_sc_dec = pl.kernel(
    _sc_dec_body,
    out_type=jax.ShapeDtypeStruct((NW, EPW), jnp.float32),
    mesh=_SC_MESH,
    scratch_types=dict(
        e0buf=pltpu.VMEM((EPW,), jnp.int32),
        e1buf=pltpu.VMEM((EPW,), jnp.int32),
        srows=pltpu.VMEM((NBUF, L, D), jnp.float32),
        drows=pltpu.VMEM((NBUF, L, D), jnp.float32),
        outbuf=pltpu.VMEM((EPW,), jnp.float32),
        sem0=pltpu.SemaphoreType.DMA((NBUF,)),
        sem1=pltpu.SemaphoreType.DMA((NBUF,)),
    ),
)


_BT = 1000


def _tc_prep_body(dp_ref, o_ref):
    deg = dp_ref[0, :, 0:1] + dp_ref[1, :, 0:1] + 1.0
    o_ref[...] = lax.rsqrt(deg)


def _tc_prep(deg_part):
    return pl.pallas_call(
        _tc_prep_body,
        grid=(N // _BT,),
        in_specs=[pl.BlockSpec((NC, _BT, L), lambda i: (0, i, 0))],
        out_specs=pl.BlockSpec((_BT, 1), lambda i: (i, 0)),
        out_shape=jax.ShapeDtypeStruct((N, 1), jnp.float32),
    )(deg_part)


def _tc_l1_body(x_ref, w_ref, dinv_ref, o_ref):
    o_ref[...] = jnp.dot(x_ref[...], w_ref[...],
                         preferred_element_type=jnp.float32) * dinv_ref[...]


def _tc_l1(x, w, dinv):
    return pl.pallas_call(
        _tc_l1_body,
        grid=(N // _BT,),
        in_specs=[
            pl.BlockSpec((_BT, D), lambda i: (i, 0)),
            pl.BlockSpec((D, D), lambda i: (0, 0)),
            pl.BlockSpec((_BT, 1), lambda i: (i, 0)),
        ],
        out_specs=pl.BlockSpec((_BT, D), lambda i: (i, 0)),
        out_shape=jax.ShapeDtypeStruct((N, D), jnp.float32),
    )(x, w, dinv)


def _tc_layer_body(p_ref, hp_ref, dinv_ref, b_ref, w_ref, o_ref):
    dinv = dinv_ref[...]
    z = (p_ref[0] + p_ref[1] + hp_ref[...]) * dinv + b_ref[...]
    z = jnp.maximum(z, 0.0)
    o_ref[...] = jnp.dot(z, w_ref[...],
                         preferred_element_type=jnp.float32) * dinv


def _tc_layer(part, hp, dinv, b, w):
    return pl.pallas_call(
        _tc_layer_body,
        grid=(N // _BT,),
        in_specs=[
            pl.BlockSpec((NC, _BT, D), lambda i: (0, i, 0)),
            pl.BlockSpec((_BT, D), lambda i: (i, 0)),
            pl.BlockSpec((_BT, 1), lambda i: (i, 0)),
            pl.BlockSpec((1, D), lambda i: (0, 0)),
            pl.BlockSpec((D, D), lambda i: (0, 0)),
        ],
        out_specs=pl.BlockSpec((_BT, D), lambda i: (i, 0)),
        out_shape=jax.ShapeDtypeStruct((N, D), jnp.float32),
    )(part, hp, dinv, b, w)


def _tc_final_body(p_ref, hp_ref, dinv_ref, b_ref, o_ref):
    o_ref[...] = ((p_ref[0] + p_ref[1] + hp_ref[...]) * dinv_ref[...]
                  + b_ref[...])


def _tc_final(part, hp, dinv, b):
    return pl.pallas_call(
        _tc_final_body,
        grid=(N // _BT,),
        in_specs=[
            pl.BlockSpec((NC, _BT, D), lambda i: (0, i, 0)),
            pl.BlockSpec((_BT, D), lambda i: (i, 0)),
            pl.BlockSpec((_BT, 1), lambda i: (i, 0)),
            pl.BlockSpec((1, D), lambda i: (0, 0)),
        ],
        out_specs=pl.BlockSpec((_BT, D), lambda i: (i, 0)),
        out_shape=jax.ShapeDtypeStruct((N, D), jnp.float32),
    )(part, hp, dinv, b)


@jax.jit
def kernel(x, edge_index, edge_label_index, W1, b1, W2, b2, W3, b3):
    src_w = edge_index[0].reshape(NW, EPW)
    dst_w = edge_index[1].reshape(NW, EPW)
    e0_w = edge_label_index[0].reshape(NW, EPW)
    e1_w = edge_label_index[1].reshape(NW, EPW)

    deg_part = _sc_deg(dst_w)
    dinv = _tc_prep(deg_part)

    hp1 = _tc_l1(x, W1, dinv)
    part1 = _sc_agg(hp1, src_w, dst_w)
    hp2 = _tc_layer(part1, hp1, dinv, b1.reshape(1, D), W2)
    part2 = _sc_agg(hp2, src_w, dst_w)
    hp3 = _tc_layer(part2, hp2, dinv, b2.reshape(1, D), W3)
    part3 = _sc_agg(hp3, src_w, dst_w)
    z3 = _tc_final(part3, hp3, dinv, b3.reshape(1, D))

    return _sc_dec(z3, e0_w, e1_w).reshape(E)


# speedup vs baseline: 16.3597x; 16.3597x over previous
import functools

import jax
import jax.numpy as jnp
from jax import lax
from jax.experimental import pallas as pl
from jax.experimental.pallas import tpu as pltpu
from jax.experimental.pallas import tpu_sc as plsc

N = 10000
E = 320000
D = 128

NC = 2
NS = 16
L = 16
NW = NC * NS
EPW = E // NW
NG = EPW // L
NBUF = 5
ROWS_PT = N // NS

_SC_MESH = plsc.VectorSubcoreMesh(core_axis_name="c", subcore_axis_name="s")


def _worker_id():
    return lax.axis_index("c") * NS + lax.axis_index("s")


def _sc_deg_body(dst_hbm, out_hbm, dstbuf, ones_v, zbuf, deg_sh, sem):
    c = lax.axis_index("c")
    s = lax.axis_index("s")
    wid = _worker_id()

    def _zero_row(i, _):
        zbuf[i, :] = jnp.zeros((L,), jnp.float32)
        return 0

    lax.fori_loop(0, ROWS_PT, _zero_row, 0)

    def _one_row(i, _):
        ones_v[i, :] = jnp.ones((L,), jnp.float32)
        return 0

    lax.fori_loop(0, L, _one_row, 0)

    pltpu.sync_copy(zbuf, deg_sh.at[pl.ds(s * ROWS_PT, ROWS_PT)])
    pltpu.sync_copy(dst_hbm.at[wid], dstbuf)
    plsc.subcore_barrier()

    def _start(j, b):
        idx = dstbuf[pl.ds(j * L, L)]
        pltpu.async_copy(ones_v, deg_sh.at[idx], sem.at[b], add=True)

    def _wait(b):
        dummy = jnp.zeros((L,), jnp.int32)
        pltpu.make_async_copy(ones_v, deg_sh.at[dummy], sem.at[b]).wait()

    for b in range(NBUF):
        _start(b, b)

    def _loop(jo, _):
        for b in range(NBUF):
            j = jo * NBUF + b
            _wait(b)
            _start(j + NBUF, b)
        return 0

    lax.fori_loop(0, NG // NBUF - 1, _loop, 0)
    for b in range(NBUF):
        _wait(b)

    plsc.subcore_barrier()
    @pl.when(s < 10)
    def _copy_out():
        pltpu.sync_copy(deg_sh.at[pl.ds(s * 1000, 1000)],
                        out_hbm.at[c, pl.ds(s * 1000, 1000)])


_sc_deg = pl.kernel(
    _sc_deg_body,
    out_type=jax.ShapeDtypeStruct((NC, N, L), jnp.float32),
    mesh=_SC_MESH,
    compiler_params=pltpu.CompilerParams(needs_layout_passes=False),
    scratch_types=dict(
        dstbuf=pltpu.VMEM((EPW,), jnp.int32),
        ones_v=pltpu.VMEM((L, L), jnp.float32),
        zbuf=pltpu.VMEM((ROWS_PT, L), jnp.float32),
        deg_sh=pltpu.VMEM_SHARED((N, L), jnp.float32),
        sem=pltpu.SemaphoreType.DMA((NBUF,)),
    ),
)


def _sc_agg_body(hp_hbm, src_hbm, dst_hbm, out_hbm,
                 srcbuf, dstbuf, rowbuf, zbuf, acc_sh, gsem):
    c = lax.axis_index("c")
    s = lax.axis_index("s")
    wid = _worker_id()

    def _zero_row(i, _):
        for k in range(D // L):
            zbuf[i, pl.ds(k * L, L)] = jnp.zeros((L,), jnp.float32)
        return 0

    lax.fori_loop(0, 25, _zero_row, 0)

    def _zero_acc(t, _):
        pltpu.sync_copy(zbuf, acc_sh.at[pl.ds(s * ROWS_PT + t * 25, 25)])
        return 0

    lax.fori_loop(0, ROWS_PT // 25, _zero_acc, 0)

    pltpu.sync_copy(src_hbm.at[wid], srcbuf)
    pltpu.sync_copy(dst_hbm.at[wid], dstbuf)
    plsc.subcore_barrier()

    def _start(j, b):
        idx = srcbuf[pl.ds(j * L, L)]
        pltpu.async_copy(hp_hbm.at[idx], rowbuf.at[b], gsem.at[b])

    def _wait(b):
        dummy = jnp.zeros((L,), jnp.int32)
        pltpu.make_async_copy(hp_hbm.at[dummy], rowbuf.at[b],
                              gsem.at[b]).wait()

    def _scatter(j, b):
        idx = dstbuf[pl.ds(j * L, L)]
        pltpu.sync_copy(rowbuf.at[b], acc_sh.at[idx], add=True)

    for b in range(NBUF):
        _start(b, b)

    def _loop(jo, _):
        for b in range(NBUF):
            j = jo * NBUF + b
            _wait(b)
            _scatter(j, b)
            _start(j + NBUF, b)
        return 0

    lax.fori_loop(0, NG // NBUF - 1, _loop, 0)
    for b in range(NBUF):
        j = NG - NBUF + b
        _wait(b)
        _scatter(j, b)

    plsc.subcore_barrier()
    @pl.when(s < 10)
    def _copy_out():
        pltpu.sync_copy(acc_sh.at[pl.ds(s * 1000, 1000)],
                        out_hbm.at[c, pl.ds(s * 1000, 1000)])


_sc_agg = pl.kernel(
    _sc_agg_body,
    out_type=jax.ShapeDtypeStruct((NC, N, D), jnp.float32),
    mesh=_SC_MESH,
    compiler_params=pltpu.CompilerParams(needs_layout_passes=False),
    scratch_types=dict(
        srcbuf=pltpu.VMEM((EPW,), jnp.int32),
        dstbuf=pltpu.VMEM((EPW,), jnp.int32),
        rowbuf=pltpu.VMEM((NBUF, L, D), jnp.float32),
        zbuf=pltpu.VMEM((25, D), jnp.float32),
        acc_sh=pltpu.VMEM_SHARED((N, D), jnp.float32),
        gsem=pltpu.SemaphoreType.DMA((NBUF,)),
    ),
)


def _sc_dec_body(z_hbm, e0_hbm, e1_hbm, out_hbm,
                 e0buf, e1buf, srows, drows, outbuf, sem0, sem1):
    wid = _worker_id()

    pltpu.sync_copy(e0_hbm.at[wid], e0buf)
    pltpu.sync_copy(e1_hbm.at[wid], e1buf)

    lanes = lax.iota(jnp.int32, L)

    def _start(j, b):
        i0 = e0buf[pl.ds(j * L, L)]
        i1 = e1buf[pl.ds(j * L, L)]
        pltpu.async_copy(z_hbm.at[i0], srows.at[b], sem0.at[b])
        pltpu.async_copy(z_hbm.at[i1], drows.at[b], sem1.at[b])

    def _wait(b):
        dummy = jnp.zeros((L,), jnp.int32)
        pltpu.make_async_copy(z_hbm.at[dummy], srows.at[b],
                              sem0.at[b]).wait()
        pltpu.make_async_copy(z_hbm.at[dummy], drows.at[b],
                              sem1.at[b]).wait()

    def _dot(j, b):
        res = jnp.zeros((L,), jnp.float32)
        for e in range(L):
            acc = srows[b, e, pl.ds(0, L)] * drows[b, e, pl.ds(0, L)]
            for k in range(1, D // L):
                acc = acc + (srows[b, e, pl.ds(k * L, L)]
                             * drows[b, e, pl.ds(k * L, L)])
            tot = jnp.sum(acc)
            res = jnp.where(lanes == e, tot, res)
        outbuf[pl.ds(j * L, L)] = res

    for b in range(NBUF):
        _start(b, b)

    def _loop(jo, _):
        for b in range(NBUF):
            j = jo * NBUF + b
            _wait(b)
            _dot(j, b)
            _start(j + NBUF, b)
        return 0

    lax.fori_loop(0, NG // NBUF - 1, _loop, 0)
    for b in range(NBUF):
        j = NG - NBUF + b
        _wait(b)
        _dot(j, b)

    pltpu.sync_copy(outbuf, out_hbm.at[wid])


_sc_dec = pl.kernel(
    _sc_dec_body,
    out_type=jax.ShapeDtypeStruct((NW, EPW), jnp.float32),
    mesh=_SC_MESH,
    compiler_params=pltpu.CompilerParams(needs_layout_passes=False),
    scratch_types=dict(
        e0buf=pltpu.VMEM((EPW,), jnp.int32),
        e1buf=pltpu.VMEM((EPW,), jnp.int32),
        srows=pltpu.VMEM((NBUF, L, D), jnp.float32),
        drows=pltpu.VMEM((NBUF, L, D), jnp.float32),
        outbuf=pltpu.VMEM((EPW,), jnp.float32),
        sem0=pltpu.SemaphoreType.DMA((NBUF,)),
        sem1=pltpu.SemaphoreType.DMA((NBUF,)),
    ),
)


_BT = 1000


def _tc_prep_body(dp_ref, o_ref):
    deg = dp_ref[0, :, 0:1] + dp_ref[1, :, 0:1] + 1.0
    o_ref[...] = lax.rsqrt(deg)


def _tc_prep(deg_part):
    return pl.pallas_call(
        _tc_prep_body,
        grid=(N // _BT,),
        in_specs=[pl.BlockSpec((NC, _BT, L), lambda i: (0, i, 0))],
        out_specs=pl.BlockSpec((_BT, 1), lambda i: (i, 0)),
        out_shape=jax.ShapeDtypeStruct((N, 1), jnp.float32),
    )(deg_part)


def _tc_l1_body(x_ref, w_ref, dinv_ref, o_ref):
    o_ref[...] = jnp.dot(x_ref[...], w_ref[...],
                         preferred_element_type=jnp.float32) * dinv_ref[...]


def _tc_l1(x, w, dinv):
    return pl.pallas_call(
        _tc_l1_body,
        grid=(N // _BT,),
        in_specs=[
            pl.BlockSpec((_BT, D), lambda i: (i, 0)),
            pl.BlockSpec((D, D), lambda i: (0, 0)),
            pl.BlockSpec((_BT, 1), lambda i: (i, 0)),
        ],
        out_specs=pl.BlockSpec((_BT, D), lambda i: (i, 0)),
        out_shape=jax.ShapeDtypeStruct((N, D), jnp.float32),
    )(x, w, dinv)


def _tc_layer_body(p_ref, hp_ref, dinv_ref, b_ref, w_ref, o_ref):
    dinv = dinv_ref[...]
    z = (p_ref[0] + p_ref[1] + hp_ref[...]) * dinv + b_ref[...]
    z = jnp.maximum(z, 0.0)
    o_ref[...] = jnp.dot(z, w_ref[...],
                         preferred_element_type=jnp.float32) * dinv


def _tc_layer(part, hp, dinv, b, w):
    return pl.pallas_call(
        _tc_layer_body,
        grid=(N // _BT,),
        in_specs=[
            pl.BlockSpec((NC, _BT, D), lambda i: (0, i, 0)),
            pl.BlockSpec((_BT, D), lambda i: (i, 0)),
            pl.BlockSpec((_BT, 1), lambda i: (i, 0)),
            pl.BlockSpec((1, D), lambda i: (0, 0)),
            pl.BlockSpec((D, D), lambda i: (0, 0)),
        ],
        out_specs=pl.BlockSpec((_BT, D), lambda i: (i, 0)),
        out_shape=jax.ShapeDtypeStruct((N, D), jnp.float32),
    )(part, hp, dinv, b, w)


def _tc_final_body(p_ref, hp_ref, dinv_ref, b_ref, o_ref):
    o_ref[...] = ((p_ref[0] + p_ref[1] + hp_ref[...]) * dinv_ref[...]
                  + b_ref[...])


def _tc_final(part, hp, dinv, b):
    return pl.pallas_call(
        _tc_final_body,
        grid=(N // _BT,),
        in_specs=[
            pl.BlockSpec((NC, _BT, D), lambda i: (0, i, 0)),
            pl.BlockSpec((_BT, D), lambda i: (i, 0)),
            pl.BlockSpec((_BT, 1), lambda i: (i, 0)),
            pl.BlockSpec((1, D), lambda i: (0, 0)),
        ],
        out_specs=pl.BlockSpec((_BT, D), lambda i: (i, 0)),
        out_shape=jax.ShapeDtypeStruct((N, D), jnp.float32),
    )(part, hp, dinv, b)


@jax.jit
def kernel(x, edge_index, edge_label_index, W1, b1, W2, b2, W3, b3):
    src_w = edge_index[0].reshape(NW, EPW)
    dst_w = edge_index[1].reshape(NW, EPW)
    e0_w = edge_label_index[0].reshape(NW, EPW)
    e1_w = edge_label_index[1].reshape(NW, EPW)

    deg_part = _sc_deg(dst_w)
    dinv = _tc_prep(deg_part)

    hp1 = _tc_l1(x, W1, dinv)
    part1 = _sc_agg(hp1, src_w, dst_w)
    hp2 = _tc_layer(part1, hp1, dinv, b1.reshape(1, D), W2)
    part2 = _sc_agg(hp2, src_w, dst_w)
    hp3 = _tc_layer(part2, hp2, dinv, b2.reshape(1, D), W3)
    part3 = _sc_agg(hp3, src_w, dst_w)
    z3 = _tc_final(part3, hp3, dinv, b3.reshape(1, D))

    return _sc_dec(z3, e0_w, e1_w).reshape(E)


# speedup vs baseline: 22.7743x; 1.3921x over previous
import functools

import jax
import jax.numpy as jnp
from jax import lax
from jax.experimental import pallas as pl
from jax.experimental.pallas import tpu as pltpu
from jax.experimental.pallas import tpu_sc as plsc

N = 10000
E = 320000
D = 128

NC = 2
NS = 16
L = 16
NW = NC * NS
EPW = E // NW
KB = 80
NBLK = EPW // KB
KBA = 40
NBLKA = EPW // KBA
RING = 3
ROWS_PT = N // NS

_SC_MESH = plsc.VectorSubcoreMesh(core_axis_name="c", subcore_axis_name="s")
_SC_PARAMS = pltpu.CompilerParams(needs_layout_passes=False)


def _worker_id():
    return lax.axis_index("c") * NS + lax.axis_index("s")


def _sc_deg_body(dst_hbm, out_hbm, dstbuf, ones_v, zbuf, deg_sh, sem):
    c = lax.axis_index("c")
    s = lax.axis_index("s")
    wid = _worker_id()

    def _zero_row(i, _):
        zbuf[i, :] = jnp.zeros((L,), jnp.float32)
        return 0

    lax.fori_loop(0, 25, _zero_row, 0)

    def _zero_deg(t, _):
        pltpu.sync_copy(zbuf, deg_sh.at[pl.ds(s * ROWS_PT + t * 25, 25)])
        return 0

    lax.fori_loop(0, ROWS_PT // 25, _zero_deg, 0)

    def _one_row(i, _):
        ones_v[i, :] = jnp.ones((L,), jnp.float32)
        return 0

    lax.fori_loop(0, KBA, _one_row, 0)

    pltpu.sync_copy(dst_hbm.at[wid], dstbuf)
    plsc.subcore_barrier()

    def _start(j):
        idx = dstbuf.at[pl.ds(j * KBA, KBA)]
        pltpu.async_copy(ones_v, deg_sh.at[idx], sem.at[lax.rem(j, RING)],
                         add=True)

    def _wait(j):
        idx = dstbuf.at[pl.ds(j * KBA, KBA)]
        pltpu.make_async_copy(ones_v, deg_sh.at[idx],
                              sem.at[lax.rem(j, RING)]).wait()

    for j in range(RING):
        _start(j)

    def _loop(j, _):
        _wait(j)

        @pl.when(j + RING < NBLKA)
        def _next():
            _start(j + RING)

        return 0

    lax.fori_loop(0, NBLKA, _loop, 0)

    plsc.subcore_barrier()
    @pl.when(s < 10)
    def _copy_out():
        pltpu.sync_copy(deg_sh.at[pl.ds(s * 1000, 1000)],
                        out_hbm.at[c, pl.ds(s * 1000, 1000)])


_sc_deg = pl.kernel(
    _sc_deg_body,
    out_type=jax.ShapeDtypeStruct((NC, N, L), jnp.float32),
    mesh=_SC_MESH,
    compiler_params=_SC_PARAMS,
    scratch_types=dict(
        dstbuf=pltpu.VMEM((EPW,), jnp.int32),
        ones_v=pltpu.VMEM((KBA, L), jnp.float32),
        zbuf=pltpu.VMEM((25, L), jnp.float32),
        deg_sh=pltpu.VMEM_SHARED((N, L), jnp.float32),
        sem=pltpu.SemaphoreType.DMA((RING,)),
    ),
)


def _sc_agg_body(hp_hbm, src_hbm, dst_hbm, out_hbm,
                 srcbuf, dstbuf, rowbuf, zbuf, acc_sh, gsem):
    c = lax.axis_index("c")
    s = lax.axis_index("s")
    wid = _worker_id()

    def _zero_row(i, _):
        for k in range(D // L):
            zbuf[i, pl.ds(k * L, L)] = jnp.zeros((L,), jnp.float32)
        return 0

    lax.fori_loop(0, 25, _zero_row, 0)

    def _zero_acc(t, _):
        pltpu.sync_copy(zbuf, acc_sh.at[pl.ds(s * ROWS_PT + t * 25, 25)])
        return 0

    lax.fori_loop(0, ROWS_PT // 25, _zero_acc, 0)

    pltpu.sync_copy(src_hbm.at[wid], srcbuf)
    pltpu.sync_copy(dst_hbm.at[wid], dstbuf)
    plsc.subcore_barrier()

    def _start(j):
        slot = lax.rem(j, RING)
        pltpu.async_copy(hp_hbm.at[srcbuf.at[pl.ds(j * KBA, KBA)]],
                         rowbuf.at[slot], gsem.at[slot])

    def _wait(j):
        slot = lax.rem(j, RING)
        pltpu.make_async_copy(hp_hbm.at[srcbuf.at[pl.ds(j * KBA, KBA)]],
                              rowbuf.at[slot], gsem.at[slot]).wait()

    def _scatter(j):
        slot = lax.rem(j, RING)
        pltpu.sync_copy(rowbuf.at[slot],
                        acc_sh.at[dstbuf.at[pl.ds(j * KBA, KBA)]], add=True)

    for j in range(RING):
        _start(j)

    def _loop(j, _):
        _wait(j)
        _scatter(j)

        @pl.when(j + RING < NBLKA)
        def _next():
            _start(j + RING)

        return 0

    lax.fori_loop(0, NBLKA, _loop, 0)

    plsc.subcore_barrier()
    @pl.when(s < 10)
    def _copy_out():
        pltpu.sync_copy(acc_sh.at[pl.ds(s * 1000, 1000)],
                        out_hbm.at[c, pl.ds(s * 1000, 1000)])


_sc_agg = pl.kernel(
    _sc_agg_body,
    out_type=jax.ShapeDtypeStruct((NC, N, D), jnp.float32),
    mesh=_SC_MESH,
    compiler_params=_SC_PARAMS,
    scratch_types=dict(
        srcbuf=pltpu.VMEM((EPW,), jnp.int32),
        dstbuf=pltpu.VMEM((EPW,), jnp.int32),
        rowbuf=pltpu.VMEM((RING, KBA, D), jnp.float32),
        zbuf=pltpu.VMEM((25, D), jnp.float32),
        acc_sh=pltpu.VMEM_SHARED((N, D), jnp.float32),
        gsem=pltpu.SemaphoreType.DMA((RING,)),
    ),
)


def _sc_dec_body(z_hbm, e0_hbm, e1_hbm, out_hbm,
                 e0buf, e1buf, srows, drows, stage, sem0, sem1):
    wid = _worker_id()
    lanes = lax.iota(jnp.int32, L)

    pltpu.sync_copy(e0_hbm.at[wid], e0buf)
    pltpu.sync_copy(e1_hbm.at[wid], e1buf)

    def _start(j):
        slot = lax.rem(j, RING)
        pltpu.async_copy(z_hbm.at[e0buf.at[pl.ds(j * KB, KB)]],
                         srows.at[slot], sem0.at[slot])
        pltpu.async_copy(z_hbm.at[e1buf.at[pl.ds(j * KB, KB)]],
                         drows.at[slot], sem1.at[slot])

    def _wait(j):
        slot = lax.rem(j, RING)
        pltpu.make_async_copy(z_hbm.at[e0buf.at[pl.ds(j * KB, KB)]],
                              srows.at[slot], sem0.at[slot]).wait()
        pltpu.make_async_copy(z_hbm.at[e1buf.at[pl.ds(j * KB, KB)]],
                              drows.at[slot], sem1.at[slot]).wait()

    def _dot(j):
        slot = lax.rem(j, RING)
        for g in range(KB // L):
            def _edge(e, res):
                r = g * L + e
                acc = (srows[slot, r, pl.ds(0, L)]
                       * drows[slot, r, pl.ds(0, L)])
                for k in range(1, D // L):
                    acc = acc + (srows[slot, r, pl.ds(k * L, L)]
                                 * drows[slot, r, pl.ds(k * L, L)])
                tot = jnp.sum(acc)
                return jnp.where(lanes == e, tot, res)

            res = lax.fori_loop(0, L, _edge, jnp.zeros((L,), jnp.float32))
            stage[pl.ds(g * L, L)] = res
        pltpu.sync_copy(stage, out_hbm.at[wid * NBLK + j])

    for j in range(RING):
        _start(j)

    def _body(j, _):
        _wait(j)
        _dot(j)

        @pl.when(j + RING < NBLK)
        def _next():
            _start(j + RING)

        return 0

    lax.fori_loop(0, NBLK, _body, 0)


_sc_dec = pl.kernel(
    _sc_dec_body,
    out_type=jax.ShapeDtypeStruct((NW * NBLK, KB), jnp.float32),
    mesh=_SC_MESH,
    compiler_params=_SC_PARAMS,
    scratch_types=dict(
        e0buf=pltpu.VMEM((EPW,), jnp.int32),
        e1buf=pltpu.VMEM((EPW,), jnp.int32),
        srows=pltpu.VMEM((RING, KB, D), jnp.float32),
        drows=pltpu.VMEM((RING, KB, D), jnp.float32),
        stage=pltpu.VMEM((KB,), jnp.float32),
        sem0=pltpu.SemaphoreType.DMA((RING,)),
        sem1=pltpu.SemaphoreType.DMA((RING,)),
    ),
)


_BT = 1000


def _tc_prep_body(dp_ref, o_ref):
    deg = dp_ref[0, :, 0:1] + dp_ref[1, :, 0:1] + 1.0
    o_ref[...] = lax.rsqrt(deg)


def _tc_prep(deg_part):
    return pl.pallas_call(
        _tc_prep_body,
        grid=(N // _BT,),
        in_specs=[pl.BlockSpec((NC, _BT, L), lambda i: (0, i, 0))],
        out_specs=pl.BlockSpec((_BT, 1), lambda i: (i, 0)),
        out_shape=jax.ShapeDtypeStruct((N, 1), jnp.float32),
    )(deg_part)


def _tc_l1_body(x_ref, w_ref, dinv_ref, o_ref):
    o_ref[...] = jnp.dot(x_ref[...], w_ref[...],
                         preferred_element_type=jnp.float32) * dinv_ref[...]


def _tc_l1(x, w, dinv):
    return pl.pallas_call(
        _tc_l1_body,
        grid=(N // _BT,),
        in_specs=[
            pl.BlockSpec((_BT, D), lambda i: (i, 0)),
            pl.BlockSpec((D, D), lambda i: (0, 0)),
            pl.BlockSpec((_BT, 1), lambda i: (i, 0)),
        ],
        out_specs=pl.BlockSpec((_BT, D), lambda i: (i, 0)),
        out_shape=jax.ShapeDtypeStruct((N, D), jnp.float32),
    )(x, w, dinv)


def _tc_layer_body(p_ref, hp_ref, dinv_ref, b_ref, w_ref, o_ref):
    dinv = dinv_ref[...]
    z = (p_ref[0] + p_ref[1] + hp_ref[...]) * dinv + b_ref[...]
    z = jnp.maximum(z, 0.0)
    o_ref[...] = jnp.dot(z, w_ref[...],
                         preferred_element_type=jnp.float32) * dinv


def _tc_layer(part, hp, dinv, b, w):
    return pl.pallas_call(
        _tc_layer_body,
        grid=(N // _BT,),
        in_specs=[
            pl.BlockSpec((NC, _BT, D), lambda i: (0, i, 0)),
            pl.BlockSpec((_BT, D), lambda i: (i, 0)),
            pl.BlockSpec((_BT, 1), lambda i: (i, 0)),
            pl.BlockSpec((1, D), lambda i: (0, 0)),
            pl.BlockSpec((D, D), lambda i: (0, 0)),
        ],
        out_specs=pl.BlockSpec((_BT, D), lambda i: (i, 0)),
        out_shape=jax.ShapeDtypeStruct((N, D), jnp.float32),
    )(part, hp, dinv, b, w)


def _tc_final_body(p_ref, hp_ref, dinv_ref, b_ref, o_ref):
    o_ref[...] = ((p_ref[0] + p_ref[1] + hp_ref[...]) * dinv_ref[...]
                  + b_ref[...])


def _tc_final(part, hp, dinv, b):
    return pl.pallas_call(
        _tc_final_body,
        grid=(N // _BT,),
        in_specs=[
            pl.BlockSpec((NC, _BT, D), lambda i: (0, i, 0)),
            pl.BlockSpec((_BT, D), lambda i: (i, 0)),
            pl.BlockSpec((_BT, 1), lambda i: (i, 0)),
            pl.BlockSpec((1, D), lambda i: (0, 0)),
        ],
        out_specs=pl.BlockSpec((_BT, D), lambda i: (i, 0)),
        out_shape=jax.ShapeDtypeStruct((N, D), jnp.float32),
    )(part, hp, dinv, b)


@jax.jit
def kernel(x, edge_index, edge_label_index, W1, b1, W2, b2, W3, b3):
    src_w = edge_index[0].reshape(NW, EPW)
    dst_w = edge_index[1].reshape(NW, EPW)
    e0_w = edge_label_index[0].reshape(NW, EPW)
    e1_w = edge_label_index[1].reshape(NW, EPW)

    deg_part = _sc_deg(dst_w)
    dinv = _tc_prep(deg_part)

    hp1 = _tc_l1(x, W1, dinv)
    part1 = _sc_agg(hp1, src_w, dst_w)
    hp2 = _tc_layer(part1, hp1, dinv, b1.reshape(1, D), W2)
    part2 = _sc_agg(hp2, src_w, dst_w)
    hp3 = _tc_layer(part2, hp2, dinv, b2.reshape(1, D), W3)
    part3 = _sc_agg(hp3, src_w, dst_w)
    z3 = _tc_final(part3, hp3, dinv, b3.reshape(1, D))

    return _sc_dec(z3, e0_w, e1_w).reshape(E)


# speedup vs baseline: 24.7823x; 1.0882x over previous
import functools

import jax
import jax.numpy as jnp
from jax import lax
from jax.experimental import pallas as pl
from jax.experimental.pallas import tpu as pltpu
from jax.experimental.pallas import tpu_sc as plsc

N = 10000
E = 320000
D = 128

NC = 2
NS = 16
L = 16
NW = NC * NS
EPW = E // NW
KB = 80
NBLK = EPW // KB
KBA = 40
NBLKA = EPW // KBA
RING = 3
SLOTS = 4
ROWS_PT = N // NS

_SC_MESH = plsc.VectorSubcoreMesh(core_axis_name="c", subcore_axis_name="s")
_SC_PARAMS = pltpu.CompilerParams(needs_layout_passes=False)


def _worker_id():
    return lax.axis_index("c") * NS + lax.axis_index("s")


def _sc_deg_body(dst_hbm, out_hbm, dstbuf, ones_v, zbuf, deg_sh, sem):
    c = lax.axis_index("c")
    s = lax.axis_index("s")
    wid = _worker_id()

    def _zero_row(i, _):
        zbuf[i, :] = jnp.zeros((L,), jnp.float32)
        return 0

    lax.fori_loop(0, 25, _zero_row, 0)

    def _zero_deg(t, _):
        pltpu.sync_copy(zbuf, deg_sh.at[pl.ds(s * ROWS_PT + t * 25, 25)])
        return 0

    lax.fori_loop(0, ROWS_PT // 25, _zero_deg, 0)

    def _one_row(i, _):
        ones_v[i, :] = jnp.ones((L,), jnp.float32)
        return 0

    lax.fori_loop(0, KBA, _one_row, 0)

    pltpu.sync_copy(dst_hbm.at[wid], dstbuf)
    plsc.subcore_barrier()

    def _start(j):
        idx = dstbuf.at[pl.ds(j * KBA, KBA)]
        pltpu.async_copy(ones_v, deg_sh.at[idx], sem.at[lax.rem(j, RING)],
                         add=True)

    def _wait(j):
        idx = dstbuf.at[pl.ds(j * KBA, KBA)]
        pltpu.make_async_copy(ones_v, deg_sh.at[idx],
                              sem.at[lax.rem(j, RING)]).wait()

    for j in range(RING):
        _start(j)

    def _loop(j, _):
        _wait(j)

        @pl.when(j + RING < NBLKA)
        def _next():
            _start(j + RING)

        return 0

    lax.fori_loop(0, NBLKA, _loop, 0)

    plsc.subcore_barrier()
    @pl.when(s < 10)
    def _copy_out():
        pltpu.sync_copy(deg_sh.at[pl.ds(s * 1000, 1000)],
                        out_hbm.at[c, pl.ds(s * 1000, 1000)])


_sc_deg = pl.kernel(
    _sc_deg_body,
    out_type=jax.ShapeDtypeStruct((NC, N, L), jnp.float32),
    mesh=_SC_MESH,
    compiler_params=_SC_PARAMS,
    scratch_types=dict(
        dstbuf=pltpu.VMEM((EPW,), jnp.int32),
        ones_v=pltpu.VMEM((KBA, L), jnp.float32),
        zbuf=pltpu.VMEM((25, L), jnp.float32),
        deg_sh=pltpu.VMEM_SHARED((N, L), jnp.float32),
        sem=pltpu.SemaphoreType.DMA((RING,)),
    ),
)


def _sc_agg_body(hp_hbm, src_hbm, dst_hbm, out_hbm,
                 srcbuf, dstbuf, rowbuf, zbuf, acc_sh, gsem):
    c = lax.axis_index("c")
    s = lax.axis_index("s")
    wid = _worker_id()

    def _zero_row(i, _):
        for k in range(D // L):
            zbuf[i, pl.ds(k * L, L)] = jnp.zeros((L,), jnp.float32)
        return 0

    lax.fori_loop(0, 25, _zero_row, 0)

    def _zero_acc(t, _):
        pltpu.sync_copy(zbuf, acc_sh.at[pl.ds(s * ROWS_PT + t * 25, 25)])
        return 0

    lax.fori_loop(0, ROWS_PT // 25, _zero_acc, 0)

    pltpu.sync_copy(src_hbm.at[wid], srcbuf)
    pltpu.sync_copy(dst_hbm.at[wid], dstbuf)
    plsc.subcore_barrier()

    def _start(j):
        slot = lax.rem(j, SLOTS)
        pltpu.async_copy(hp_hbm.at[srcbuf.at[pl.ds(j * KBA, KBA)]],
                         rowbuf.at[slot], gsem.at[slot])

    def _wait(j):
        slot = lax.rem(j, SLOTS)
        pltpu.make_async_copy(hp_hbm.at[srcbuf.at[pl.ds(j * KBA, KBA)]],
                              rowbuf.at[slot], gsem.at[slot]).wait()

    def _scatter(j):
        slot = lax.rem(j, SLOTS)
        pltpu.sync_copy(rowbuf.at[slot],
                        acc_sh.at[dstbuf.at[pl.ds(j * KBA, KBA)]], add=True)

    for j in range(RING):
        _start(j)

    def _loop(j, _):
        _wait(j)

        @pl.when(j + RING < NBLKA)
        def _next():
            _start(j + RING)

        _scatter(j)
        return 0

    lax.fori_loop(0, NBLKA, _loop, 0)

    plsc.subcore_barrier()
    @pl.when(s < 10)
    def _copy_out():
        pltpu.sync_copy(acc_sh.at[pl.ds(s * 1000, 1000)],
                        out_hbm.at[c, pl.ds(s * 1000, 1000)])


_sc_agg = pl.kernel(
    _sc_agg_body,
    out_type=jax.ShapeDtypeStruct((NC, N, D), jnp.float32),
    mesh=_SC_MESH,
    compiler_params=_SC_PARAMS,
    scratch_types=dict(
        srcbuf=pltpu.VMEM((EPW,), jnp.int32),
        dstbuf=pltpu.VMEM((EPW,), jnp.int32),
        rowbuf=pltpu.VMEM((SLOTS, KBA, D), jnp.float32),
        zbuf=pltpu.VMEM((25, D), jnp.float32),
        acc_sh=pltpu.VMEM_SHARED((N, D), jnp.float32),
        gsem=pltpu.SemaphoreType.DMA((SLOTS,)),
    ),
)


def _sc_dec_body(z_hbm, e0_hbm, e1_hbm, out_hbm,
                 e0buf, e1buf, srows, drows, stage, sem0, sem1):
    wid = _worker_id()
    lanes = lax.iota(jnp.int32, L)

    pltpu.sync_copy(e0_hbm.at[wid], e0buf)
    pltpu.sync_copy(e1_hbm.at[wid], e1buf)

    def _start(j):
        slot = lax.rem(j, SLOTS)
        pltpu.async_copy(z_hbm.at[e0buf.at[pl.ds(j * KB, KB)]],
                         srows.at[slot], sem0.at[slot])
        pltpu.async_copy(z_hbm.at[e1buf.at[pl.ds(j * KB, KB)]],
                         drows.at[slot], sem1.at[slot])

    def _wait(j):
        slot = lax.rem(j, SLOTS)
        pltpu.make_async_copy(z_hbm.at[e0buf.at[pl.ds(j * KB, KB)]],
                              srows.at[slot], sem0.at[slot]).wait()
        pltpu.make_async_copy(z_hbm.at[e1buf.at[pl.ds(j * KB, KB)]],
                              drows.at[slot], sem1.at[slot]).wait()

    def _dot(j):
        slot = lax.rem(j, SLOTS)
        for g in range(KB // L):
            def _edge(e, res):
                r = g * L + e
                acc = (srows[slot, r, pl.ds(0, L)]
                       * drows[slot, r, pl.ds(0, L)])
                for k in range(1, D // L):
                    acc = acc + (srows[slot, r, pl.ds(k * L, L)]
                                 * drows[slot, r, pl.ds(k * L, L)])
                tot = jnp.sum(acc)
                return jnp.where(lanes == e, tot, res)

            res = lax.fori_loop(0, L, _edge, jnp.zeros((L,), jnp.float32))
            stage[pl.ds(g * L, L)] = res
        pltpu.sync_copy(stage, out_hbm.at[wid * NBLK + j])

    for j in range(RING):
        _start(j)

    def _body(j, _):
        _wait(j)

        @pl.when(j + RING < NBLK)
        def _next():
            _start(j + RING)

        _dot(j)
        return 0

    lax.fori_loop(0, NBLK, _body, 0)


_sc_dec = pl.kernel(
    _sc_dec_body,
    out_type=jax.ShapeDtypeStruct((NW * NBLK, KB), jnp.float32),
    mesh=_SC_MESH,
    compiler_params=_SC_PARAMS,
    scratch_types=dict(
        e0buf=pltpu.VMEM((EPW,), jnp.int32),
        e1buf=pltpu.VMEM((EPW,), jnp.int32),
        srows=pltpu.VMEM((SLOTS, KB, D), jnp.float32),
        drows=pltpu.VMEM((SLOTS, KB, D), jnp.float32),
        stage=pltpu.VMEM((KB,), jnp.float32),
        sem0=pltpu.SemaphoreType.DMA((SLOTS,)),
        sem1=pltpu.SemaphoreType.DMA((SLOTS,)),
    ),
)


_BT = 1000


def _tc_prep_body(dp_ref, o_ref):
    deg = dp_ref[0, :, 0:1] + dp_ref[1, :, 0:1] + 1.0
    o_ref[...] = lax.rsqrt(deg)


def _tc_prep(deg_part):
    return pl.pallas_call(
        _tc_prep_body,
        grid=(N // _BT,),
        in_specs=[pl.BlockSpec((NC, _BT, L), lambda i: (0, i, 0))],
        out_specs=pl.BlockSpec((_BT, 1), lambda i: (i, 0)),
        out_shape=jax.ShapeDtypeStruct((N, 1), jnp.float32),
    )(deg_part)


def _tc_l1_body(x_ref, w_ref, dinv_ref, o_ref):
    o_ref[...] = jnp.dot(x_ref[...], w_ref[...],
                         preferred_element_type=jnp.float32) * dinv_ref[...]


def _tc_l1(x, w, dinv):
    return pl.pallas_call(
        _tc_l1_body,
        grid=(N // _BT,),
        in_specs=[
            pl.BlockSpec((_BT, D), lambda i: (i, 0)),
            pl.BlockSpec((D, D), lambda i: (0, 0)),
            pl.BlockSpec((_BT, 1), lambda i: (i, 0)),
        ],
        out_specs=pl.BlockSpec((_BT, D), lambda i: (i, 0)),
        out_shape=jax.ShapeDtypeStruct((N, D), jnp.float32),
    )(x, w, dinv)


def _tc_layer_body(p_ref, hp_ref, dinv_ref, b_ref, w_ref, o_ref):
    dinv = dinv_ref[...]
    z = (p_ref[0] + p_ref[1] + hp_ref[...]) * dinv + b_ref[...]
    z = jnp.maximum(z, 0.0)
    o_ref[...] = jnp.dot(z, w_ref[...],
                         preferred_element_type=jnp.float32) * dinv


def _tc_layer(part, hp, dinv, b, w):
    return pl.pallas_call(
        _tc_layer_body,
        grid=(N // _BT,),
        in_specs=[
            pl.BlockSpec((NC, _BT, D), lambda i: (0, i, 0)),
            pl.BlockSpec((_BT, D), lambda i: (i, 0)),
            pl.BlockSpec((_BT, 1), lambda i: (i, 0)),
            pl.BlockSpec((1, D), lambda i: (0, 0)),
            pl.BlockSpec((D, D), lambda i: (0, 0)),
        ],
        out_specs=pl.BlockSpec((_BT, D), lambda i: (i, 0)),
        out_shape=jax.ShapeDtypeStruct((N, D), jnp.float32),
    )(part, hp, dinv, b, w)


def _tc_final_body(p_ref, hp_ref, dinv_ref, b_ref, o_ref):
    o_ref[...] = ((p_ref[0] + p_ref[1] + hp_ref[...]) * dinv_ref[...]
                  + b_ref[...])


def _tc_final(part, hp, dinv, b):
    return pl.pallas_call(
        _tc_final_body,
        grid=(N // _BT,),
        in_specs=[
            pl.BlockSpec((NC, _BT, D), lambda i: (0, i, 0)),
            pl.BlockSpec((_BT, D), lambda i: (i, 0)),
            pl.BlockSpec((_BT, 1), lambda i: (i, 0)),
            pl.BlockSpec((1, D), lambda i: (0, 0)),
        ],
        out_specs=pl.BlockSpec((_BT, D), lambda i: (i, 0)),
        out_shape=jax.ShapeDtypeStruct((N, D), jnp.float32),
    )(part, hp, dinv, b)


@jax.jit
def kernel(x, edge_index, edge_label_index, W1, b1, W2, b2, W3, b3):
    src_w = edge_index[0].reshape(NW, EPW)
    dst_w = edge_index[1].reshape(NW, EPW)
    e0_w = edge_label_index[0].reshape(NW, EPW)
    e1_w = edge_label_index[1].reshape(NW, EPW)

    deg_part = _sc_deg(dst_w)
    dinv = _tc_prep(deg_part)

    hp1 = _tc_l1(x, W1, dinv)
    part1 = _sc_agg(hp1, src_w, dst_w)
    hp2 = _tc_layer(part1, hp1, dinv, b1.reshape(1, D), W2)
    part2 = _sc_agg(hp2, src_w, dst_w)
    hp3 = _tc_layer(part2, hp2, dinv, b2.reshape(1, D), W3)
    part3 = _sc_agg(hp3, src_w, dst_w)
    z3 = _tc_final(part3, hp3, dinv, b3.reshape(1, D))

    return _sc_dec(z3, e0_w, e1_w).reshape(E)


# speedup vs baseline: 25.0257x; 1.0098x over previous
import functools

import jax
import jax.numpy as jnp
from jax import lax
from jax.experimental import pallas as pl
from jax.experimental.pallas import tpu as pltpu
from jax.experimental.pallas import tpu_sc as plsc

N = 10000
E = 320000
D = 128

NC = 2
NS = 16
L = 16
NW = NC * NS
EPW = E // NW
KB = 80
NBLK = EPW // KB
KBA = 40
NBLKA = EPW // KBA
RING = 3
SLOTS = 4
ROWS_PT = N // NS

_SC_MESH = plsc.VectorSubcoreMesh(core_axis_name="c", subcore_axis_name="s")
_SC_PARAMS = pltpu.CompilerParams(needs_layout_passes=False)


def _worker_id():
    return lax.axis_index("c") * NS + lax.axis_index("s")


def _sc_deg_body(dst_hbm, out_hbm, dstbuf, ones_v, zbuf, deg_sh, sem):
    c = lax.axis_index("c")
    s = lax.axis_index("s")
    wid = _worker_id()

    def _zero_row(i, _):
        zbuf[i, :] = jnp.zeros((L,), jnp.float32)
        return 0

    lax.fori_loop(0, 25, _zero_row, 0)

    def _zero_deg(t, _):
        pltpu.sync_copy(zbuf, deg_sh.at[pl.ds(s * ROWS_PT + t * 25, 25)])
        return 0

    lax.fori_loop(0, ROWS_PT // 25, _zero_deg, 0)

    def _one_row(i, _):
        ones_v[i, :] = jnp.ones((L,), jnp.float32)
        return 0

    lax.fori_loop(0, KBA, _one_row, 0)

    pltpu.sync_copy(dst_hbm.at[wid], dstbuf)
    plsc.subcore_barrier()

    def _start(j):
        idx = dstbuf.at[pl.ds(j * KBA, KBA)]
        pltpu.async_copy(ones_v, deg_sh.at[idx], sem.at[lax.rem(j, RING)],
                         add=True)

    def _wait(j):
        idx = dstbuf.at[pl.ds(j * KBA, KBA)]
        pltpu.make_async_copy(ones_v, deg_sh.at[idx],
                              sem.at[lax.rem(j, RING)]).wait()

    for j in range(RING):
        _start(j)

    def _loop(j, _):
        _wait(j)

        @pl.when(j + RING < NBLKA)
        def _next():
            _start(j + RING)

        return 0

    lax.fori_loop(0, NBLKA, _loop, 0)

    plsc.subcore_barrier()
    @pl.when(s < 10)
    def _copy_out():
        pltpu.sync_copy(deg_sh.at[pl.ds(s * 1000, 1000)],
                        out_hbm.at[c, pl.ds(s * 1000, 1000)])


_sc_deg = pl.kernel(
    _sc_deg_body,
    out_type=jax.ShapeDtypeStruct((NC, N, L), jnp.float32),
    mesh=_SC_MESH,
    compiler_params=_SC_PARAMS,
    scratch_types=dict(
        dstbuf=pltpu.VMEM((EPW,), jnp.int32),
        ones_v=pltpu.VMEM((KBA, L), jnp.float32),
        zbuf=pltpu.VMEM((25, L), jnp.float32),
        deg_sh=pltpu.VMEM_SHARED((N, L), jnp.float32),
        sem=pltpu.SemaphoreType.DMA((RING,)),
    ),
)


def _sc_agg_body(hp_hbm, src_hbm, dst_hbm, out_hbm,
                 srcbuf, dstbuf, rowbuf, zbuf, acc_sh, gsem):
    c = lax.axis_index("c")
    s = lax.axis_index("s")
    wid = _worker_id()

    def _zero_row(i, _):
        for k in range(D // L):
            zbuf[i, pl.ds(k * L, L)] = jnp.zeros((L,), jnp.float32)
        return 0

    lax.fori_loop(0, 25, _zero_row, 0)

    def _zero_acc(t, _):
        pltpu.sync_copy(zbuf, acc_sh.at[pl.ds(s * ROWS_PT + t * 25, 25)])
        return 0

    lax.fori_loop(0, ROWS_PT // 25, _zero_acc, 0)

    pltpu.sync_copy(src_hbm.at[wid], srcbuf)
    pltpu.sync_copy(dst_hbm.at[wid], dstbuf)
    plsc.subcore_barrier()

    def _start(j):
        slot = lax.rem(j, SLOTS)
        pltpu.async_copy(hp_hbm.at[srcbuf.at[pl.ds(j * KBA, KBA)]],
                         rowbuf.at[slot], gsem.at[slot])

    def _wait(j):
        slot = lax.rem(j, SLOTS)
        pltpu.make_async_copy(hp_hbm.at[srcbuf.at[pl.ds(j * KBA, KBA)]],
                              rowbuf.at[slot], gsem.at[slot]).wait()

    def _scatter(j):
        slot = lax.rem(j, SLOTS)
        pltpu.sync_copy(rowbuf.at[slot],
                        acc_sh.at[dstbuf.at[pl.ds(j * KBA, KBA)]], add=True)

    for j in range(RING):
        _start(j)

    def _loop(j, _):
        _wait(j)

        @pl.when(j + RING < NBLKA)
        def _next():
            _start(j + RING)

        _scatter(j)
        return 0

    lax.fori_loop(0, NBLKA, _loop, 0)

    plsc.subcore_barrier()
    @pl.when(s < 10)
    def _copy_out():
        pltpu.sync_copy(acc_sh.at[pl.ds(s * 1000, 1000)],
                        out_hbm.at[c, pl.ds(s * 1000, 1000)])


_sc_agg = pl.kernel(
    _sc_agg_body,
    out_type=jax.ShapeDtypeStruct((NC, N, D), jnp.float32),
    mesh=_SC_MESH,
    compiler_params=_SC_PARAMS,
    scratch_types=dict(
        srcbuf=pltpu.VMEM((EPW,), jnp.int32),
        dstbuf=pltpu.VMEM((EPW,), jnp.int32),
        rowbuf=pltpu.VMEM((SLOTS, KBA, D), jnp.float32),
        zbuf=pltpu.VMEM((25, D), jnp.float32),
        acc_sh=pltpu.VMEM_SHARED((N, D), jnp.float32),
        gsem=pltpu.SemaphoreType.DMA((SLOTS,)),
    ),
)


def _sc_dec_body(z_hbm, e0_hbm, e1_hbm, out_hbm,
                 e0buf, e1buf, srows, drows, stage, sem0, sem1):
    wid = _worker_id()
    lanes = lax.iota(jnp.int32, L)

    pltpu.sync_copy(e0_hbm.at[wid], e0buf)
    pltpu.sync_copy(e1_hbm.at[wid], e1buf)

    def _start(j):
        slot = lax.rem(j, SLOTS)
        pltpu.async_copy(z_hbm.at[e0buf.at[pl.ds(j * KB, KB)]],
                         srows.at[slot], sem0.at[slot])
        pltpu.async_copy(z_hbm.at[e1buf.at[pl.ds(j * KB, KB)]],
                         drows.at[slot], sem1.at[slot])

    def _wait(j):
        slot = lax.rem(j, SLOTS)
        pltpu.make_async_copy(z_hbm.at[e0buf.at[pl.ds(j * KB, KB)]],
                              srows.at[slot], sem0.at[slot]).wait()
        pltpu.make_async_copy(z_hbm.at[e1buf.at[pl.ds(j * KB, KB)]],
                              drows.at[slot], sem1.at[slot]).wait()

    def _dot(j):
        slot = lax.rem(j, SLOTS)
        for g in range(KB // L):
            def _edge2(e2, res):
                for u in range(2):
                    e = e2 * 2 + u
                    r = g * L + e
                    acc = (srows[slot, r, pl.ds(0, L)]
                           * drows[slot, r, pl.ds(0, L)])
                    for k in range(1, D // L):
                        acc = acc + (srows[slot, r, pl.ds(k * L, L)]
                                     * drows[slot, r, pl.ds(k * L, L)])
                    tot = jnp.sum(acc)
                    res = jnp.where(lanes == e, tot, res)
                return res

            res = lax.fori_loop(0, L // 2, _edge2,
                                jnp.zeros((L,), jnp.float32))
            stage[pl.ds(g * L, L)] = res
        pltpu.sync_copy(stage, out_hbm.at[wid * NBLK + j])

    for j in range(RING):
        _start(j)

    def _body(j, _):
        _wait(j)

        @pl.when(j + RING < NBLK)
        def _next():
            _start(j + RING)

        _dot(j)
        return 0

    lax.fori_loop(0, NBLK, _body, 0)


_sc_dec = pl.kernel(
    _sc_dec_body,
    out_type=jax.ShapeDtypeStruct((NW * NBLK, KB), jnp.float32),
    mesh=_SC_MESH,
    compiler_params=_SC_PARAMS,
    scratch_types=dict(
        e0buf=pltpu.VMEM((EPW,), jnp.int32),
        e1buf=pltpu.VMEM((EPW,), jnp.int32),
        srows=pltpu.VMEM((SLOTS, KB, D), jnp.float32),
        drows=pltpu.VMEM((SLOTS, KB, D), jnp.float32),
        stage=pltpu.VMEM((KB,), jnp.float32),
        sem0=pltpu.SemaphoreType.DMA((SLOTS,)),
        sem1=pltpu.SemaphoreType.DMA((SLOTS,)),
    ),
)


_BT = 1000


def _tc_mm_body(x_ref, w_ref, o_ref):
    o_ref[...] = jnp.dot(x_ref[...], w_ref[...],
                         preferred_element_type=jnp.float32)


def _tc_mm(x, w):
    return pl.pallas_call(
        _tc_mm_body,
        grid=(N // _BT,),
        in_specs=[
            pl.BlockSpec((_BT, D), lambda i: (i, 0)),
            pl.BlockSpec((D, D), lambda i: (0, 0)),
        ],
        out_specs=pl.BlockSpec((_BT, D), lambda i: (i, 0)),
        out_shape=jax.ShapeDtypeStruct((N, D), jnp.float32),
    )(x, w)


def _tc_prep_body(dp_ref, mm_ref, dinv_ref, hp_ref):
    deg = dp_ref[0, :, 0:1] + dp_ref[1, :, 0:1] + 1.0
    dinv = lax.rsqrt(deg)
    dinv_ref[...] = dinv
    hp_ref[...] = mm_ref[...] * dinv


def _tc_prep(deg_part, mm1):
    return pl.pallas_call(
        _tc_prep_body,
        grid=(N // _BT,),
        in_specs=[
            pl.BlockSpec((NC, _BT, L), lambda i: (0, i, 0)),
            pl.BlockSpec((_BT, D), lambda i: (i, 0)),
        ],
        out_specs=[
            pl.BlockSpec((_BT, 1), lambda i: (i, 0)),
            pl.BlockSpec((_BT, D), lambda i: (i, 0)),
        ],
        out_shape=[
            jax.ShapeDtypeStruct((N, 1), jnp.float32),
            jax.ShapeDtypeStruct((N, D), jnp.float32),
        ],
    )(deg_part, mm1)


def _tc_layer_body(p_ref, hp_ref, dinv_ref, b_ref, w_ref, o_ref):
    dinv = dinv_ref[...]
    z = (p_ref[0] + p_ref[1] + hp_ref[...]) * dinv + b_ref[...]
    z = jnp.maximum(z, 0.0)
    o_ref[...] = jnp.dot(z, w_ref[...],
                         preferred_element_type=jnp.float32) * dinv


def _tc_layer(part, hp, dinv, b, w):
    return pl.pallas_call(
        _tc_layer_body,
        grid=(N // _BT,),
        in_specs=[
            pl.BlockSpec((NC, _BT, D), lambda i: (0, i, 0)),
            pl.BlockSpec((_BT, D), lambda i: (i, 0)),
            pl.BlockSpec((_BT, 1), lambda i: (i, 0)),
            pl.BlockSpec((1, D), lambda i: (0, 0)),
            pl.BlockSpec((D, D), lambda i: (0, 0)),
        ],
        out_specs=pl.BlockSpec((_BT, D), lambda i: (i, 0)),
        out_shape=jax.ShapeDtypeStruct((N, D), jnp.float32),
    )(part, hp, dinv, b, w)


def _tc_final_body(p_ref, hp_ref, dinv_ref, b_ref, o_ref):
    o_ref[...] = ((p_ref[0] + p_ref[1] + hp_ref[...]) * dinv_ref[...]
                  + b_ref[...])


def _tc_final(part, hp, dinv, b):
    return pl.pallas_call(
        _tc_final_body,
        grid=(N // _BT,),
        in_specs=[
            pl.BlockSpec((NC, _BT, D), lambda i: (0, i, 0)),
            pl.BlockSpec((_BT, D), lambda i: (i, 0)),
            pl.BlockSpec((_BT, 1), lambda i: (i, 0)),
            pl.BlockSpec((1, D), lambda i: (0, 0)),
        ],
        out_specs=pl.BlockSpec((_BT, D), lambda i: (i, 0)),
        out_shape=jax.ShapeDtypeStruct((N, D), jnp.float32),
    )(part, hp, dinv, b)


@jax.jit
def kernel(x, edge_index, edge_label_index, W1, b1, W2, b2, W3, b3):
    src_w = edge_index[0].reshape(NW, EPW)
    dst_w = edge_index[1].reshape(NW, EPW)
    e0_w = edge_label_index[0].reshape(NW, EPW)
    e1_w = edge_label_index[1].reshape(NW, EPW)

    mm1 = _tc_mm(x, W1)
    deg_part = _sc_deg(dst_w)
    dinv, hp1 = _tc_prep(deg_part, mm1)
    part1 = _sc_agg(hp1, src_w, dst_w)
    hp2 = _tc_layer(part1, hp1, dinv, b1.reshape(1, D), W2)
    part2 = _sc_agg(hp2, src_w, dst_w)
    hp3 = _tc_layer(part2, hp2, dinv, b2.reshape(1, D), W3)
    part3 = _sc_agg(hp3, src_w, dst_w)
    z3 = _tc_final(part3, hp3, dinv, b3.reshape(1, D))

    return _sc_dec(z3, e0_w, e1_w).reshape(E)


# speedup vs baseline: 25.3260x; 1.0120x over previous
import functools

import jax
import jax.numpy as jnp
from jax import lax
from jax.experimental import pallas as pl
from jax.experimental.pallas import tpu as pltpu
from jax.experimental.pallas import tpu_sc as plsc

N = 10000
E = 320000
D = 128

NC = 2
NS = 16
L = 16
NW = NC * NS
EPW = E // NW
KB = 80
NBLK = EPW // KB
KBA = 40
NBLKA = EPW // KBA
RING = 3
SLOTS = 4
ROWS_PT = N // NS

_SC_MESH = plsc.VectorSubcoreMesh(core_axis_name="c", subcore_axis_name="s")
_SC_PARAMS = pltpu.CompilerParams(needs_layout_passes=False)


def _worker_id():
    return lax.axis_index("c") * NS + lax.axis_index("s")


def _sc_deg_body(dst_hbm, out_hbm, dstbuf, ones_v, zbuf, deg_sh, sem):
    c = lax.axis_index("c")
    s = lax.axis_index("s")
    wid = _worker_id()

    def _zero_row(i, _):
        zbuf[i, :] = jnp.zeros((L,), jnp.float32)
        return 0

    lax.fori_loop(0, 25, _zero_row, 0)

    def _zero_deg(t, _):
        pltpu.sync_copy(zbuf, deg_sh.at[pl.ds(s * ROWS_PT + t * 25, 25)])
        return 0

    lax.fori_loop(0, ROWS_PT // 25, _zero_deg, 0)

    def _one_row(i, _):
        ones_v[i, :] = jnp.ones((L,), jnp.float32)
        return 0

    lax.fori_loop(0, KBA, _one_row, 0)

    pltpu.sync_copy(dst_hbm.at[wid], dstbuf)
    plsc.subcore_barrier()

    def _start(j):
        idx = dstbuf.at[pl.ds(j * KBA, KBA)]
        pltpu.async_copy(ones_v, deg_sh.at[idx], sem.at[lax.rem(j, RING)],
                         add=True)

    def _wait(j):
        idx = dstbuf.at[pl.ds(j * KBA, KBA)]
        pltpu.make_async_copy(ones_v, deg_sh.at[idx],
                              sem.at[lax.rem(j, RING)]).wait()

    for j in range(RING):
        _start(j)

    def _loop(j, _):
        _wait(j)

        @pl.when(j + RING < NBLKA)
        def _next():
            _start(j + RING)

        return 0

    lax.fori_loop(0, NBLKA, _loop, 0)

    plsc.subcore_barrier()
    @pl.when(s < 10)
    def _copy_out():
        pltpu.sync_copy(deg_sh.at[pl.ds(s * 1000, 1000)],
                        out_hbm.at[c, pl.ds(s * 1000, 1000)])


_sc_deg = pl.kernel(
    _sc_deg_body,
    out_type=jax.ShapeDtypeStruct((NC, N, L), jnp.float32),
    mesh=_SC_MESH,
    compiler_params=_SC_PARAMS,
    scratch_types=dict(
        dstbuf=pltpu.VMEM((EPW,), jnp.int32),
        ones_v=pltpu.VMEM((KBA, L), jnp.float32),
        zbuf=pltpu.VMEM((25, L), jnp.float32),
        deg_sh=pltpu.VMEM_SHARED((N, L), jnp.float32),
        sem=pltpu.SemaphoreType.DMA((RING,)),
    ),
)


def _sc_agg_body(hp_hbm, src_hbm, dst_hbm, out_hbm,
                 srcbuf, dstbuf, rowbuf, zbuf, acc_sh, gsem):
    c = lax.axis_index("c")
    s = lax.axis_index("s")
    wid = _worker_id()

    def _zero_row(i, _):
        for k in range(D // L):
            zbuf[i, pl.ds(k * L, L)] = jnp.zeros((L,), jnp.float32)
        return 0

    lax.fori_loop(0, 25, _zero_row, 0)

    def _zero_acc(t, _):
        pltpu.sync_copy(zbuf, acc_sh.at[pl.ds(s * ROWS_PT + t * 25, 25)])
        return 0

    lax.fori_loop(0, ROWS_PT // 25, _zero_acc, 0)

    pltpu.sync_copy(src_hbm.at[wid], srcbuf)
    pltpu.sync_copy(dst_hbm.at[wid], dstbuf)
    plsc.subcore_barrier()

    def _start(j):
        slot = lax.rem(j, SLOTS)
        pltpu.async_copy(hp_hbm.at[srcbuf.at[pl.ds(j * KBA, KBA)]],
                         rowbuf.at[slot], gsem.at[slot])

    def _wait(j):
        slot = lax.rem(j, SLOTS)
        pltpu.make_async_copy(hp_hbm.at[srcbuf.at[pl.ds(j * KBA, KBA)]],
                              rowbuf.at[slot], gsem.at[slot]).wait()

    def _scatter(j):
        slot = lax.rem(j, SLOTS)
        pltpu.sync_copy(rowbuf.at[slot],
                        acc_sh.at[dstbuf.at[pl.ds(j * KBA, KBA)]], add=True)

    for j in range(RING):
        _start(j)

    def _loop(j, _):
        _wait(j)

        @pl.when(j + RING < NBLKA)
        def _next():
            _start(j + RING)

        _scatter(j)
        return 0

    lax.fori_loop(0, NBLKA, _loop, 0)

    plsc.subcore_barrier()
    @pl.when(s < 10)
    def _copy_out():
        pltpu.sync_copy(acc_sh.at[pl.ds(s * 1000, 1000)],
                        out_hbm.at[c, pl.ds(s * 1000, 1000)])


_sc_agg = pl.kernel(
    _sc_agg_body,
    out_type=jax.ShapeDtypeStruct((NC, N, D), jnp.float32),
    mesh=_SC_MESH,
    compiler_params=_SC_PARAMS,
    scratch_types=dict(
        srcbuf=pltpu.VMEM((EPW,), jnp.int32),
        dstbuf=pltpu.VMEM((EPW,), jnp.int32),
        rowbuf=pltpu.VMEM((SLOTS, KBA, D), jnp.float32),
        zbuf=pltpu.VMEM((25, D), jnp.float32),
        acc_sh=pltpu.VMEM_SHARED((N, D), jnp.float32),
        gsem=pltpu.SemaphoreType.DMA((SLOTS,)),
    ),
)


def _sc_dec_body(z_hbm, e0_hbm, e1_hbm, out_hbm,
                 e0buf, e1buf, srows, drows, stage, sem0, sem1):
    wid = _worker_id()
    lanes = lax.iota(jnp.int32, L)

    pltpu.sync_copy(e0_hbm.at[wid], e0buf)
    pltpu.sync_copy(e1_hbm.at[wid], e1buf)

    def _start(j):
        slot = lax.rem(j, SLOTS)
        pltpu.async_copy(z_hbm.at[e0buf.at[pl.ds(j * KB, KB)]],
                         srows.at[slot], sem0.at[slot])
        pltpu.async_copy(z_hbm.at[e1buf.at[pl.ds(j * KB, KB)]],
                         drows.at[slot], sem1.at[slot])

    def _wait(j):
        slot = lax.rem(j, SLOTS)
        pltpu.make_async_copy(z_hbm.at[e0buf.at[pl.ds(j * KB, KB)]],
                              srows.at[slot], sem0.at[slot]).wait()
        pltpu.make_async_copy(z_hbm.at[e1buf.at[pl.ds(j * KB, KB)]],
                              drows.at[slot], sem1.at[slot]).wait()

    def _dot(j):
        slot = lax.rem(j, SLOTS)
        for g in range(KB // L):
            def _edge2(e2, res):
                for u in range(2):
                    e = e2 * 2 + u
                    r = g * L + e
                    acc = jnp.zeros((L,), jnp.float32)
                    for k in range(D // (2 * L)):
                        sw = srows[slot, r, pl.ds(k * L, L)]
                        dw = drows[slot, r, pl.ds(k * L, L)]
                        sab = plsc.bitcast(sw, jnp.bfloat16)
                        dab = plsc.bitcast(dw, jnp.bfloat16)
                        sa, sb = plsc.unpack(
                            sab, format=plsc.PackFormat.INTERLEAVED)
                        da, db = plsc.unpack(
                            dab, format=plsc.PackFormat.INTERLEAVED)
                        acc = acc + sa * da + sb * db
                    tot = jnp.sum(acc)
                    res = jnp.where(lanes == e, tot, res)
                return res

            res = lax.fori_loop(0, L // 2, _edge2,
                                jnp.zeros((L,), jnp.float32))
            stage[pl.ds(g * L, L)] = res
        pltpu.sync_copy(stage, out_hbm.at[wid * NBLK + j])

    for j in range(RING):
        _start(j)

    def _body(j, _):
        _wait(j)

        @pl.when(j + RING < NBLK)
        def _next():
            _start(j + RING)

        _dot(j)
        return 0

    lax.fori_loop(0, NBLK, _body, 0)


_sc_dec = pl.kernel(
    _sc_dec_body,
    out_type=jax.ShapeDtypeStruct((NW * NBLK, KB), jnp.float32),
    mesh=_SC_MESH,
    compiler_params=pltpu.CompilerParams(needs_layout_passes=False,
                                         use_tc_tiling_on_sc=False),
    scratch_types=dict(
        e0buf=pltpu.VMEM((EPW,), jnp.int32),
        e1buf=pltpu.VMEM((EPW,), jnp.int32),
        srows=pltpu.VMEM((SLOTS, KB, D // 2), jnp.int32),
        drows=pltpu.VMEM((SLOTS, KB, D // 2), jnp.int32),
        stage=pltpu.VMEM((KB,), jnp.float32),
        sem0=pltpu.SemaphoreType.DMA((SLOTS,)),
        sem1=pltpu.SemaphoreType.DMA((SLOTS,)),
    ),
)


_BT = 1000


def _tc_mm_body(x_ref, w_ref, o_ref):
    o_ref[...] = jnp.dot(x_ref[...], w_ref[...],
                         preferred_element_type=jnp.float32)


def _tc_mm(x, w):
    return pl.pallas_call(
        _tc_mm_body,
        grid=(N // _BT,),
        in_specs=[
            pl.BlockSpec((_BT, D), lambda i: (i, 0)),
            pl.BlockSpec((D, D), lambda i: (0, 0)),
        ],
        out_specs=pl.BlockSpec((_BT, D), lambda i: (i, 0)),
        out_shape=jax.ShapeDtypeStruct((N, D), jnp.float32),
    )(x, w)


def _tc_prep_body(dp_ref, mm_ref, dinv_ref, hp_ref):
    deg = dp_ref[0, :, 0:1] + dp_ref[1, :, 0:1] + 1.0
    dinv = lax.rsqrt(deg)
    dinv_ref[...] = dinv
    hp_ref[...] = mm_ref[...] * dinv


def _tc_prep(deg_part, mm1):
    return pl.pallas_call(
        _tc_prep_body,
        grid=(N // _BT,),
        in_specs=[
            pl.BlockSpec((NC, _BT, L), lambda i: (0, i, 0)),
            pl.BlockSpec((_BT, D), lambda i: (i, 0)),
        ],
        out_specs=[
            pl.BlockSpec((_BT, 1), lambda i: (i, 0)),
            pl.BlockSpec((_BT, D), lambda i: (i, 0)),
        ],
        out_shape=[
            jax.ShapeDtypeStruct((N, 1), jnp.float32),
            jax.ShapeDtypeStruct((N, D), jnp.float32),
        ],
    )(deg_part, mm1)


def _tc_layer_body(p_ref, hp_ref, dinv_ref, b_ref, w_ref, o_ref):
    dinv = dinv_ref[...]
    z = (p_ref[0] + p_ref[1] + hp_ref[...]) * dinv + b_ref[...]
    z = jnp.maximum(z, 0.0)
    o_ref[...] = jnp.dot(z, w_ref[...],
                         preferred_element_type=jnp.float32) * dinv


def _tc_layer(part, hp, dinv, b, w):
    return pl.pallas_call(
        _tc_layer_body,
        grid=(N // _BT,),
        in_specs=[
            pl.BlockSpec((NC, _BT, D), lambda i: (0, i, 0)),
            pl.BlockSpec((_BT, D), lambda i: (i, 0)),
            pl.BlockSpec((_BT, 1), lambda i: (i, 0)),
            pl.BlockSpec((1, D), lambda i: (0, 0)),
            pl.BlockSpec((D, D), lambda i: (0, 0)),
        ],
        out_specs=pl.BlockSpec((_BT, D), lambda i: (i, 0)),
        out_shape=jax.ShapeDtypeStruct((N, D), jnp.float32),
    )(part, hp, dinv, b, w)


def _tc_final_body(p_ref, hp_ref, dinv_ref, b_ref, o_ref):
    z3 = ((p_ref[0] + p_ref[1] + hp_ref[...]) * dinv_ref[...]
          + b_ref[...])
    o_ref[...] = z3.astype(jnp.bfloat16)


def _tc_final(part, hp, dinv, b):
    return pl.pallas_call(
        _tc_final_body,
        grid=(N // _BT,),
        in_specs=[
            pl.BlockSpec((NC, _BT, D), lambda i: (0, i, 0)),
            pl.BlockSpec((_BT, D), lambda i: (i, 0)),
            pl.BlockSpec((_BT, 1), lambda i: (i, 0)),
            pl.BlockSpec((1, D), lambda i: (0, 0)),
        ],
        out_specs=pl.BlockSpec((_BT, D), lambda i: (i, 0)),
        out_shape=jax.ShapeDtypeStruct((N, D), jnp.bfloat16),
    )(part, hp, dinv, b)


@jax.jit
def kernel(x, edge_index, edge_label_index, W1, b1, W2, b2, W3, b3):
    src_w = edge_index[0].reshape(NW, EPW)
    dst_w = edge_index[1].reshape(NW, EPW)
    e0_w = edge_label_index[0].reshape(NW, EPW)
    e1_w = edge_label_index[1].reshape(NW, EPW)

    mm1 = _tc_mm(x, W1)
    deg_part = _sc_deg(dst_w)
    dinv, hp1 = _tc_prep(deg_part, mm1)
    part1 = _sc_agg(hp1, src_w, dst_w)
    hp2 = _tc_layer(part1, hp1, dinv, b1.reshape(1, D), W2)
    part2 = _sc_agg(hp2, src_w, dst_w)
    hp3 = _tc_layer(part2, hp2, dinv, b2.reshape(1, D), W3)
    part3 = _sc_agg(hp3, src_w, dst_w)
    z3 = _tc_final(part3, hp3, dinv, b3.reshape(1, D))
    z3w = jax.lax.bitcast_convert_type(
        z3.reshape(N, D // 2, 2), jnp.int32)

    return _sc_dec(z3w, e0_w, e1_w).reshape(E)
